# Initial kernel scaffold; baseline (speedup 1.0000x reference)
import jax
import jax.numpy as jnp
from jax.experimental import pallas as pl


def kernel(node_states, edges, edge_weights, pos_cls, W_pre, b_pre, kernels, attn_kernels, W_out, b_out):
    raise NotImplementedError("write your pallas kernel here")


# speedup vs baseline: 53.6538x; 53.6538x over previous
import functools

import jax
import jax.numpy as jnp
from jax import lax
from jax.experimental import pallas as pl
from jax.experimental.pallas import tpu as pltpu
from jax.experimental.pallas import tpu_sc as plsc

N = 10000
E = 320000
D = 128
HID = 16
HEADS = 8
H = HID * HEADS
OUT = 64

NC = 2
NS = 16
NW = NC * NS
CE = E // NW
G = 80
NCHUNK = CE // G
RPT = 624
RPT_EXTRA = N - NS * RPT
ZR = 16

_f32 = jnp.float32
_i32 = jnp.int32


TCBLK = 1000


def _tc_stage_a(ns_ref, wpre_ref, bpre_ref, k_ref, a_ref, x_ref, nt_ref, al_ref):
    x = jnp.maximum(
        jnp.dot(ns_ref[...], wpre_ref[...], preferred_element_type=_f32)
        + bpre_ref[...], 0.0)
    x_ref[...] = x
    nt = jnp.dot(x, k_ref[...], preferred_element_type=_f32)
    nt_ref[...] = nt
    al_ref[...] = jnp.dot(nt, a_ref[...], preferred_element_type=_f32)


def _tc_stage_b(parts_ref, xp_ref, k_ref, a_ref, x_ref, nt_ref, al_ref):
    xn = jnp.maximum(parts_ref[0] + parts_ref[1], 0.0) + xp_ref[...]
    x_ref[...] = xn
    nt = jnp.dot(xn, k_ref[...], preferred_element_type=_f32)
    nt_ref[...] = nt
    al_ref[...] = jnp.dot(nt, a_ref[...], preferred_element_type=_f32)


def _tc_stage_c(parts_ref, xp_ref, wout_ref, bout_ref, o_ref):
    xn = jnp.maximum(parts_ref[0] + parts_ref[1], 0.0) + xp_ref[...]
    o_ref[...] = (jnp.dot(xn, wout_ref[...], preferred_element_type=_f32)
                  + bout_ref[...])


def _run_stage_a(ns, wpre, bpre, kcat, a12):
    grid = (N // TCBLK,)
    return pl.pallas_call(
        _tc_stage_a,
        grid=grid,
        in_specs=[
            pl.BlockSpec((TCBLK, D), lambda i: (i, 0)),
            pl.BlockSpec((D, H), lambda i: (0, 0)),
            pl.BlockSpec((1, H), lambda i: (0, 0)),
            pl.BlockSpec((H, H), lambda i: (0, 0)),
            pl.BlockSpec((H, 2 * HEADS), lambda i: (0, 0)),
        ],
        out_specs=[
            pl.BlockSpec((TCBLK, H), lambda i: (i, 0)),
            pl.BlockSpec((TCBLK, H), lambda i: (i, 0)),
            pl.BlockSpec((TCBLK, 2 * HEADS), lambda i: (i, 0)),
        ],
        out_shape=[
            jax.ShapeDtypeStruct((N, H), _f32),
            jax.ShapeDtypeStruct((N, H), _f32),
            jax.ShapeDtypeStruct((N, 2 * HEADS), _f32),
        ],
    )(ns, wpre, bpre, kcat, a12)


def _run_stage_b(parts, xprev, kcat, a12):
    grid = (N // TCBLK,)
    return pl.pallas_call(
        _tc_stage_b,
        grid=grid,
        in_specs=[
            pl.BlockSpec((2, TCBLK, H), lambda i: (0, i, 0)),
            pl.BlockSpec((TCBLK, H), lambda i: (i, 0)),
            pl.BlockSpec((H, H), lambda i: (0, 0)),
            pl.BlockSpec((H, 2 * HEADS), lambda i: (0, 0)),
        ],
        out_specs=[
            pl.BlockSpec((TCBLK, H), lambda i: (i, 0)),
            pl.BlockSpec((TCBLK, H), lambda i: (i, 0)),
            pl.BlockSpec((TCBLK, 2 * HEADS), lambda i: (i, 0)),
        ],
        out_shape=[
            jax.ShapeDtypeStruct((N, H), _f32),
            jax.ShapeDtypeStruct((N, H), _f32),
            jax.ShapeDtypeStruct((N, 2 * HEADS), _f32),
        ],
    )(parts, xprev, kcat, a12)


def _run_stage_c(parts, xprev, wout, bout):
    grid = (N // TCBLK,)
    return pl.pallas_call(
        _tc_stage_c,
        grid=grid,
        in_specs=[
            pl.BlockSpec((2, TCBLK, H), lambda i: (0, i, 0)),
            pl.BlockSpec((TCBLK, H), lambda i: (i, 0)),
            pl.BlockSpec((H, OUT), lambda i: (0, 0)),
            pl.BlockSpec((1, OUT), lambda i: (0, 0)),
        ],
        out_specs=pl.BlockSpec((TCBLK, OUT), lambda i: (i, 0)),
        out_shape=jax.ShapeDtypeStruct((N, OUT), _f32),
    )(parts, xprev, wout, bout)


_MESH = plsc.VectorSubcoreMesh(core_axis_name="c", subcore_axis_name="s")
_LANE_IOTA = None


def _lanes():
    return lax.iota(_i32, 16)


def _splat(vec, j):
    dnums = lax.GatherDimensionNumbers(
        offset_dims=(), collapsed_slice_dims=(0,), start_index_map=(0,))
    idx = jnp.full((16, 1), j, dtype=_i32)
    return lax.gather(vec, idx, dnums, (1,),
                      mode=lax.GatherScatterMode.PROMISE_IN_BOUNDS)


def _sc_scores(src_hbm, dst_hbm, ew_hbm, alpha_hbm, zrows_hbm,
               p_hbm, ssum_hbm,
               idx_s, idx_d, ew_v, a_s, a_d, p_v, zbuf, ssum_sp,
               sem_a, sem_b):
    c = lax.axis_index("c")
    s = lax.axis_index("s")
    wid = c * NS + s

    pltpu.sync_copy(zrows_hbm, zbuf)

    def zero_blk(z, _):
        pltpu.sync_copy(zbuf, ssum_sp.at[pl.ds(s * RPT + z * ZR, ZR), :])
        return 0

    nzero = (RPT + jnp.where(s == NS - 1, RPT_EXTRA, 0)) // ZR
    lax.fori_loop(0, nzero, zero_blk, 0)
    plsc.subcore_barrier()

    pltpu.sync_copy(src_hbm.at[wid], idx_s)
    pltpu.sync_copy(dst_hbm.at[wid], idx_d)
    pltpu.sync_copy(ew_hbm.at[wid], ew_v)

    def chunk(cg, _):
        pltpu.async_copy(alpha_hbm.at[idx_s.at[cg]], a_s, sem_a).wait()
        pltpu.async_copy(alpha_hbm.at[idx_d.at[cg]], a_d, sem_b).wait()

        def grp(i, _):
            g0 = i * 16
            rows = g0 + _lanes()
            ewv = ew_v[cg, pl.ds(g0, 16)]
            for h in range(HEADS):
                sv = plsc.load_gather(a_s, [rows, jnp.full((16,), h, _i32)])
                dv = plsc.load_gather(a_d, [rows, jnp.full((16,), h + 8, _i32)])
                sc = ewv * (sv + dv)
                sc = jnp.where(sc >= 0.0, sc, 0.01 * sc)
                sc = jnp.clip(sc, -2.0, 2.0)
                pv = jnp.exp(sc)
                plsc.store_scatter(p_v, [rows, jnp.full((16,), h, _i32)], pv)
            return 0

        lax.fori_loop(0, G // 16, grp, 0)
        pltpu.sync_copy(p_v, p_hbm.at[wid, cg])
        pltpu.sync_copy(p_v, ssum_sp.at[idx_s.at[cg]], add=True)
        return 0

    lax.fori_loop(0, NCHUNK, chunk, 0)
    plsc.subcore_barrier()
    pltpu.sync_copy(ssum_sp.at[pl.ds(s * RPT, RPT), :],
                    ssum_hbm.at[c, pl.ds(s * RPT, RPT), :])

    @pl.when(s == NS - 1)
    def _():
        pltpu.sync_copy(ssum_sp.at[pl.ds(NS * RPT, RPT_EXTRA), :],
                        ssum_hbm.at[c, pl.ds(NS * RPT, RPT_EXTRA), :])


def _sc_aggregate(src_hbm, dst_hbm, p_hbm, s0_hbm, s1_hbm, nt_hbm, zrows_hbm,
                  out_hbm,
                  idx_s, idx_d, p_v, s0_v, s1_v, nrm_f, rows_v, zbuf, out_sp,
                  sem_r, sem_a, sem_b):
    c = lax.axis_index("c")
    s = lax.axis_index("s")
    wid = c * NS + s

    pltpu.sync_copy(zrows_hbm, zbuf)

    def zero_blk(z, _):
        pltpu.sync_copy(zbuf, out_sp.at[pl.ds(s * RPT + z * ZR, ZR), :])
        return 0

    nzero = (RPT + jnp.where(s == NS - 1, RPT_EXTRA, 0)) // ZR
    lax.fori_loop(0, nzero, zero_blk, 0)
    plsc.subcore_barrier()

    pltpu.sync_copy(src_hbm.at[wid], idx_s)
    pltpu.sync_copy(dst_hbm.at[wid], idx_d)

    def chunk(cg, _):
        rows_cp = pltpu.async_copy(nt_hbm.at[idx_d.at[cg]], rows_v, sem_r)
        pltpu.async_copy(s0_hbm.at[idx_s.at[cg]], s0_v, sem_a).wait()
        pltpu.async_copy(s1_hbm.at[idx_s.at[cg]], s1_v, sem_b).wait()
        pltpu.sync_copy(p_hbm.at[wid, cg], p_v)

        def nrm_grp(i, _):
            flat = i * 16 + _lanes()
            r = flat // 8
            col = flat % 8
            pv = plsc.load_gather(p_v, [r, col])
            t0 = plsc.load_gather(s0_v, [r, col])
            t1 = plsc.load_gather(s1_v, [r, col])
            plsc.store_scatter(nrm_f, [flat], pv / (t0 + t1))
            return 0

        lax.fori_loop(0, (G * HEADS) // 16, nrm_grp, 0)
        rows_cp.wait()

        def pair(i, _):
            nv = nrm_f[pl.ds(i * 16, 16)]
            e0 = i * 2
            for j in range(HEADS):
                sl0 = _splat(nv, j)
                rows_v[e0, pl.ds(j * HID, HID)] = (
                    rows_v[e0, pl.ds(j * HID, HID)] * sl0)
                sl1 = _splat(nv, 8 + j)
                rows_v[e0 + 1, pl.ds(j * HID, HID)] = (
                    rows_v[e0 + 1, pl.ds(j * HID, HID)] * sl1)
            return 0

        lax.fori_loop(0, G // 2, pair, 0)
        pltpu.sync_copy(rows_v, out_sp.at[idx_s.at[cg]], add=True)
        return 0

    lax.fori_loop(0, NCHUNK, chunk, 0)
    plsc.subcore_barrier()
    pltpu.sync_copy(out_sp.at[pl.ds(s * RPT, RPT), :],
                    out_hbm.at[c, pl.ds(s * RPT, RPT), :])

    @pl.when(s == NS - 1)
    def _():
        pltpu.sync_copy(out_sp.at[pl.ds(NS * RPT, RPT_EXTRA), :],
                        out_hbm.at[c, pl.ds(NS * RPT, RPT_EXTRA), :])


def _run_sc_scores(src3, dst3, ew3, alpha, zrows8):
    return pl.kernel(
        _sc_scores,
        out_type=[
            jax.ShapeDtypeStruct((NW, NCHUNK, G, HEADS), _f32),
            jax.ShapeDtypeStruct((NC, N, HEADS), _f32),
        ],
        mesh=_MESH,
        compiler_params=pltpu.CompilerParams(needs_layout_passes=False, use_tc_tiling_on_sc=False),
        scratch_types=[
            pltpu.VMEM((NCHUNK, G), _i32),
            pltpu.VMEM((NCHUNK, G), _i32),
            pltpu.VMEM((NCHUNK, G), _f32),
            pltpu.VMEM((G, 2 * HEADS), _f32),
            pltpu.VMEM((G, 2 * HEADS), _f32),
            pltpu.VMEM((G, HEADS), _f32),
            pltpu.VMEM((ZR, HEADS), _f32),
            pltpu.VMEM_SHARED((N, HEADS), _f32),
            pltpu.SemaphoreType.DMA,
            pltpu.SemaphoreType.DMA,
        ],
    )(src3, dst3, ew3, alpha, zrows8)


def _run_sc_aggregate(src3, dst3, p, s0, s1, nt, zrows128):
    return pl.kernel(
        _sc_aggregate,
        out_type=jax.ShapeDtypeStruct((NC, N, H), _f32),
        mesh=_MESH,
        compiler_params=pltpu.CompilerParams(needs_layout_passes=False, use_tc_tiling_on_sc=False),
        scratch_types=[
            pltpu.VMEM((NCHUNK, G), _i32),
            pltpu.VMEM((NCHUNK, G), _i32),
            pltpu.VMEM((G, HEADS), _f32),
            pltpu.VMEM((G, HEADS), _f32),
            pltpu.VMEM((G, HEADS), _f32),
            pltpu.VMEM((G * HEADS,), _f32),
            pltpu.VMEM((G, H), _f32),
            pltpu.VMEM((ZR, H), _f32),
            pltpu.VMEM_SHARED((N, H), _f32),
            pltpu.SemaphoreType.DMA,
            pltpu.SemaphoreType.DMA,
            pltpu.SemaphoreType.DMA,
        ],
    )(src3, dst3, p, s0, s1, nt, zrows128)


def kernel(node_states, edges, edge_weights, pos_cls, W_pre, b_pre, kernels,
           attn_kernels, W_out, b_out):
    ns = node_states[0]
    src = edges[0, :, 0]
    dst = edges[0, :, 1]
    ew = edge_weights[0]

    src3 = src.reshape(NW, NCHUNK, G)
    dst3 = dst.reshape(NW, NCHUNK, G)
    ew3 = ew.reshape(NW, NCHUNK, G)

    eye = jnp.eye(HEADS, dtype=_f32)
    a12 = []
    kcat = []
    for l in range(2):
        a1 = attn_kernels[l, :, :HID, 0]
        a2 = attn_kernels[l, :, HID:, 0]
        A1 = (eye[:, None, :] * a1[:, :, None]).reshape(H, HEADS)
        A2 = (eye[:, None, :] * a2[:, :, None]).reshape(H, HEADS)
        a12.append(jnp.concatenate([A1, A2], axis=1))
        kcat.append(kernels[l].transpose(1, 0, 2).reshape(H, H))

    zrows8 = jnp.zeros((ZR, HEADS), _f32)
    zrows128 = jnp.zeros((ZR, H), _f32)

    x0, nt0, al0 = _run_stage_a(ns, W_pre, b_pre.reshape(1, H), kcat[0], a12[0])

    p0, ss0 = _run_sc_scores(src3, dst3, ew3, al0, zrows8)
    parts0 = _run_sc_aggregate(src3, dst3, p0, ss0[0], ss0[1], nt0, zrows128)

    x1, nt1, al1 = _run_stage_b(parts0, x0, kcat[1], a12[1])

    p1, ss1 = _run_sc_scores(src3, dst3, ew3, al1, zrows8)
    parts1 = _run_sc_aggregate(src3, dst3, p1, ss1[0], ss1[1], nt1, zrows128)

    out = _run_stage_c(parts1, x1, W_out, b_out.reshape(1, OUT))
    return out[None, :, :]


# speedup vs baseline: 123.7917x; 2.3072x over previous
import functools

import jax
import jax.numpy as jnp
from jax import lax
from jax.experimental import pallas as pl
from jax.experimental.pallas import tpu as pltpu
from jax.experimental.pallas import tpu_sc as plsc

N = 10000
E = 320000
D = 128
HID = 16
HEADS = 8
H = HID * HEADS
OUT = 64

NC = 2
NS = 16
NW = NC * NS
CE = E // NW
G = 80
NCHUNK = CE // G
RPT = 624
RPT_EXTRA = N - NS * RPT
ZR = 16
ZRB = 8

_f32 = jnp.float32
_i32 = jnp.int32


TCBLK = 1000


def _tc_stage_a(ns_ref, wpre_ref, bpre_ref, k_ref, a_ref, x_ref, nt_ref, al_ref):
    x = jnp.maximum(
        jnp.dot(ns_ref[...], wpre_ref[...], preferred_element_type=_f32)
        + bpre_ref[...], 0.0)
    x_ref[...] = x
    nt = jnp.dot(x, k_ref[...], preferred_element_type=_f32)
    nt_ref[...] = nt
    al_ref[...] = jnp.dot(nt, a_ref[...], preferred_element_type=_f32)


def _tc_stage_b(parts_ref, xp_ref, k_ref, a_ref, x_ref, nt_ref, al_ref):
    xn = jnp.maximum(parts_ref[0] + parts_ref[1], 0.0) + xp_ref[...]
    x_ref[...] = xn
    nt = jnp.dot(xn, k_ref[...], preferred_element_type=_f32)
    nt_ref[...] = nt
    al_ref[...] = jnp.dot(nt, a_ref[...], preferred_element_type=_f32)


def _tc_stage_c(parts_ref, xp_ref, wout_ref, bout_ref, o_ref):
    xn = jnp.maximum(parts_ref[0] + parts_ref[1], 0.0) + xp_ref[...]
    o_ref[...] = (jnp.dot(xn, wout_ref[...], preferred_element_type=_f32)
                  + bout_ref[...])


def _run_stage_a(ns, wpre, bpre, kcat, a12):
    grid = (N // TCBLK,)
    return pl.pallas_call(
        _tc_stage_a,
        grid=grid,
        in_specs=[
            pl.BlockSpec((TCBLK, D), lambda i: (i, 0)),
            pl.BlockSpec((D, H), lambda i: (0, 0)),
            pl.BlockSpec((1, H), lambda i: (0, 0)),
            pl.BlockSpec((H, H), lambda i: (0, 0)),
            pl.BlockSpec((H, 2 * HEADS), lambda i: (0, 0)),
        ],
        out_specs=[
            pl.BlockSpec((TCBLK, H), lambda i: (i, 0)),
            pl.BlockSpec((TCBLK, H), lambda i: (i, 0)),
            pl.BlockSpec((TCBLK, 2 * HEADS), lambda i: (i, 0)),
        ],
        out_shape=[
            jax.ShapeDtypeStruct((N, H), _f32),
            jax.ShapeDtypeStruct((N, H), _f32),
            jax.ShapeDtypeStruct((N, 2 * HEADS), _f32),
        ],
    )(ns, wpre, bpre, kcat, a12)


def _run_stage_b(parts, xprev, kcat, a12):
    grid = (N // TCBLK,)
    return pl.pallas_call(
        _tc_stage_b,
        grid=grid,
        in_specs=[
            pl.BlockSpec((2, TCBLK, H), lambda i: (0, i, 0)),
            pl.BlockSpec((TCBLK, H), lambda i: (i, 0)),
            pl.BlockSpec((H, H), lambda i: (0, 0)),
            pl.BlockSpec((H, 2 * HEADS), lambda i: (0, 0)),
        ],
        out_specs=[
            pl.BlockSpec((TCBLK, H), lambda i: (i, 0)),
            pl.BlockSpec((TCBLK, H), lambda i: (i, 0)),
            pl.BlockSpec((TCBLK, 2 * HEADS), lambda i: (i, 0)),
        ],
        out_shape=[
            jax.ShapeDtypeStruct((N, H), _f32),
            jax.ShapeDtypeStruct((N, H), _f32),
            jax.ShapeDtypeStruct((N, 2 * HEADS), _f32),
        ],
    )(parts, xprev, kcat, a12)


def _run_stage_c(parts, xprev, wout, bout):
    grid = (N // TCBLK,)
    return pl.pallas_call(
        _tc_stage_c,
        grid=grid,
        in_specs=[
            pl.BlockSpec((2, TCBLK, H), lambda i: (0, i, 0)),
            pl.BlockSpec((TCBLK, H), lambda i: (i, 0)),
            pl.BlockSpec((H, OUT), lambda i: (0, 0)),
            pl.BlockSpec((1, OUT), lambda i: (0, 0)),
        ],
        out_specs=pl.BlockSpec((TCBLK, OUT), lambda i: (i, 0)),
        out_shape=jax.ShapeDtypeStruct((N, OUT), _f32),
    )(parts, xprev, wout, bout)


_MESH = plsc.VectorSubcoreMesh(core_axis_name="c", subcore_axis_name="s")
_LANE_IOTA = None


def _lanes():
    return lax.iota(_i32, 16)


def _splat(vec, j):
    dnums = lax.GatherDimensionNumbers(
        offset_dims=(), collapsed_slice_dims=(0,), start_index_map=(0,))
    idx = jnp.full((16, 1), j, dtype=_i32)
    return lax.gather(vec, idx, dnums, (1,),
                      mode=lax.GatherScatterMode.PROMISE_IN_BOUNDS)


def _sc_scores(src_hbm, dst_hbm, ew_hbm, alpha_hbm, zrows_hbm,
               p_hbm, ssum_hbm,
               idx_s, idx_d, ew_v, a_s0, a_s1, a_d0, a_d1, p_v0, p_v1,
               zbuf, ssum_sp,
               sga0, sga1, sgd0, sgd1, swp0, swp1, sws0, sws1):
    c = lax.axis_index("c")
    s = lax.axis_index("s")
    wid = c * NS + s
    a_s = [a_s0, a_s1]
    a_d = [a_d0, a_d1]
    p_v = [p_v0, p_v1]
    sga = [sga0, sga1]
    sgd = [sgd0, sgd1]
    swp = [swp0, swp1]
    sws = [sws0, sws1]

    pltpu.sync_copy(zrows_hbm, zbuf)

    def zero_blk(z, _):
        pltpu.sync_copy(zbuf, ssum_sp.at[pl.ds(s * RPT + z * ZR, ZR), :])
        return 0

    nzero = (RPT + jnp.where(s == NS - 1, RPT_EXTRA, 0)) // ZR
    lax.fori_loop(0, nzero, zero_blk, 0)
    plsc.subcore_barrier()

    pltpu.sync_copy(src_hbm.at[wid], idx_s)
    pltpu.sync_copy(dst_hbm.at[wid], idx_d)
    pltpu.sync_copy(ew_hbm.at[wid], ew_v)

    def issue_gathers(cg, b):
        pltpu.async_copy(alpha_hbm.at[idx_s.at[cg]], a_s[b], sga[b])
        pltpu.async_copy(alpha_hbm.at[idx_d.at[cg]], a_d[b], sgd[b])

    def wait_gathers(cg, b):
        pltpu.make_async_copy(alpha_hbm.at[idx_s.at[cg]], a_s[b], sga[b]).wait()
        pltpu.make_async_copy(alpha_hbm.at[idx_d.at[cg]], a_d[b], sgd[b]).wait()

    def wait_stores(cg, b):
        pltpu.make_async_copy(p_v[b], p_hbm.at[wid, cg], swp[b]).wait()
        pltpu.make_async_copy(p_v[b], ssum_sp.at[idx_s.at[cg]], sws[b]).wait()

    def compute(cg, b):
        def grp(i, _):
            g0 = i * 16
            rows = g0 + _lanes()
            ewv = ew_v[cg, pl.ds(g0, 16)]
            for h in range(HEADS):
                sv = plsc.load_gather(a_s[b], [rows, jnp.full((16,), h, _i32)])
                dv = plsc.load_gather(a_d[b],
                                      [rows, jnp.full((16,), h + 8, _i32)])
                sc = ewv * (sv + dv)
                sc = jnp.where(sc >= 0.0, sc, 0.01 * sc)
                sc = jnp.clip(sc, -2.0, 2.0)
                pv = jnp.exp(sc)
                plsc.store_scatter(p_v[b], [rows, jnp.full((16,), h, _i32)], pv)
            return 0

        lax.fori_loop(0, G // 16, grp, 0)
        pltpu.async_copy(p_v[b], p_hbm.at[wid, cg], swp[b])
        pltpu.async_copy(p_v[b], ssum_sp.at[idx_s.at[cg]], sws[b], add=True)

    issue_gathers(0, 0)

    def pair(i, _):
        cg0 = i * 2
        issue_gathers(cg0 + 1, 1)
        wait_gathers(cg0, 0)

        @pl.when(i > 0)
        def _():
            wait_stores(cg0, 0)

        compute(cg0, 0)
        issue_gathers(cg0 + 2, 0)
        wait_gathers(cg0 + 1, 1)

        @pl.when(i > 0)
        def _():
            wait_stores(cg0 + 1, 1)

        compute(cg0 + 1, 1)
        return 0

    lax.fori_loop(0, (NCHUNK - 1) // 2, pair, 0)
    wait_gathers(NCHUNK - 1, 0)
    wait_stores(NCHUNK - 1, 0)
    compute(NCHUNK - 1, 0)
    wait_stores(NCHUNK - 2, 1)
    wait_stores(NCHUNK - 1, 0)
    plsc.subcore_barrier()
    pltpu.sync_copy(ssum_sp.at[pl.ds(s * RPT, RPT), :],
                    ssum_hbm.at[c, pl.ds(s * RPT, RPT), :])

    @pl.when(s == NS - 1)
    def _():
        pltpu.sync_copy(ssum_sp.at[pl.ds(NS * RPT, RPT_EXTRA), :],
                        ssum_hbm.at[c, pl.ds(NS * RPT, RPT_EXTRA), :])


def _sc_aggregate(src_hbm, dst_hbm, p_hbm, s0_hbm, s1_hbm, nt_hbm, zrows_hbm,
                  out_hbm,
                  ixs0, ixs1, ixs2, ixs3, ixd0, ixd1, ixd2, ixd3,
                  p_v0, p_v1, s0_v0, s0_v1, s1_v0, s1_v1,
                  nrm_f, rin0, rin1, rout0, rout1, zbuf, out_sp,
                  six0, six1, six2, six3,
                  sr0, sr1, ss00, ss01, ss10, ss11, sp0, sp1, sw0, sw1):
    c = lax.axis_index("c")
    s = lax.axis_index("s")
    wid = c * NS + s
    ixs = [ixs0, ixs1, ixs2, ixs3]
    ixd = [ixd0, ixd1, ixd2, ixd3]
    six = [six0, six1, six2, six3]
    p_v = [p_v0, p_v1]
    s0_v = [s0_v0, s0_v1]
    s1_v = [s1_v0, s1_v1]
    rin = [rin0, rin1]
    rout = [rout0, rout1]
    sr = [sr0, sr1]
    ss0 = [ss00, ss01]
    ss1 = [ss10, ss11]
    sp = [sp0, sp1]
    sw = [sw0, sw1]

    pltpu.sync_copy(zrows_hbm, zbuf)

    def zero_blk(z, _):
        pltpu.sync_copy(zbuf, out_sp.at[pl.ds(s * RPT + z * ZRB, ZRB), :])
        return 0

    nzero = (RPT + jnp.where(s == NS - 1, RPT_EXTRA, 0)) // ZRB
    lax.fori_loop(0, nzero, zero_blk, 0)
    plsc.subcore_barrier()

    def issue_idx(cg, k):
        pltpu.async_copy(src_hbm.at[wid, cg], ixs[k], six[k])
        pltpu.async_copy(dst_hbm.at[wid, cg], ixd[k], six[k])

    def wait_idx(cg, k):
        pltpu.make_async_copy(src_hbm.at[wid, cg], ixs[k], six[k]).wait()
        pltpu.make_async_copy(dst_hbm.at[wid, cg], ixd[k], six[k]).wait()

    def issue_gathers(cg, k, b):
        pltpu.async_copy(nt_hbm.at[ixd[k]], rin[b], sr[b])
        pltpu.async_copy(s0_hbm.at[ixs[k]], s0_v[b], ss0[b])
        pltpu.async_copy(s1_hbm.at[ixs[k]], s1_v[b], ss1[b])
        pltpu.async_copy(p_hbm.at[wid, cg], p_v[b], sp[b])

    def wait_gathers(cg, k, b):
        pltpu.make_async_copy(nt_hbm.at[ixd[k]], rin[b], sr[b]).wait()
        pltpu.make_async_copy(s0_hbm.at[ixs[k]], s0_v[b], ss0[b]).wait()
        pltpu.make_async_copy(s1_hbm.at[ixs[k]], s1_v[b], ss1[b]).wait()
        pltpu.make_async_copy(p_hbm.at[wid, cg], p_v[b], sp[b]).wait()

    def wait_store(k, b):
        pltpu.make_async_copy(rout[b], out_sp.at[ixs[k]], sw[b]).wait()

    def compute(k, b):
        def nrm_grp(i, _):
            flat = i * 16 + _lanes()
            r = flat // 8
            col = flat % 8
            pv = plsc.load_gather(p_v[b], [r, col])
            t0 = plsc.load_gather(s0_v[b], [r, col])
            t1 = plsc.load_gather(s1_v[b], [r, col])
            plsc.store_scatter(nrm_f, [flat], pv / (t0 + t1))
            return 0

        lax.fori_loop(0, (G * HEADS) // 16, nrm_grp, 0)

        def pair(i, _):
            nv = nrm_f[pl.ds(i * 16, 16)]
            e0 = i * 2
            for j in range(HEADS):
                sl0 = _splat(nv, j)
                rout[b][e0, pl.ds(j * HID, HID)] = (
                    rin[b][e0, pl.ds(j * HID, HID)] * sl0)
                sl1 = _splat(nv, 8 + j)
                rout[b][e0 + 1, pl.ds(j * HID, HID)] = (
                    rin[b][e0 + 1, pl.ds(j * HID, HID)] * sl1)
            return 0

        lax.fori_loop(0, G // 2, pair, 0)
        pltpu.async_copy(rout[b], out_sp.at[ixs[k]], sw[b], add=True)

    issue_idx(0, 0)
    issue_idx(1, 1)
    wait_idx(0, 0)
    issue_gathers(0, 0, 0)

    def quad(i, _):
        for kk in range(4):
            cg = i * 4 + kk
            k4 = kk
            b2 = kk % 2
            if kk >= 2:
                wait_store((kk - 2) % 4, b2)
            else:
                @pl.when(i > 0)
                def _():
                    wait_store((kk + 2) % 4, b2)

            @pl.when(cg + 2 < NCHUNK)
            def _():
                issue_idx(cg + 2, (kk + 2) % 4)

            wait_idx(cg + 1, (kk + 1) % 4)
            issue_gathers(cg + 1, (kk + 1) % 4, 1 - b2)
            wait_gathers(cg, k4, b2)
            compute(k4, b2)
        return 0

    lax.fori_loop(0, NCHUNK // 4, quad, 0)
    wait_store(2, 0)
    wait_gathers(NCHUNK - 1, 0, 0)
    compute(0, 0)
    wait_store(3, 1)
    wait_store(0, 0)
    plsc.subcore_barrier()
    pltpu.sync_copy(out_sp.at[pl.ds(s * RPT, RPT), :],
                    out_hbm.at[c, pl.ds(s * RPT, RPT), :])

    @pl.when(s == NS - 1)
    def _():
        pltpu.sync_copy(out_sp.at[pl.ds(NS * RPT, RPT_EXTRA), :],
                        out_hbm.at[c, pl.ds(NS * RPT, RPT_EXTRA), :])


def _run_sc_scores(src3, dst3, ew3, alpha, zrows8):
    return pl.kernel(
        _sc_scores,
        out_type=[
            jax.ShapeDtypeStruct((NW, NCHUNK, G, HEADS), _f32),
            jax.ShapeDtypeStruct((NC, N, HEADS), _f32),
        ],
        mesh=_MESH,
        compiler_params=pltpu.CompilerParams(needs_layout_passes=False, use_tc_tiling_on_sc=False),
        scratch_types=[
            pltpu.VMEM((NCHUNK, G), _i32),
            pltpu.VMEM((NCHUNK, G), _i32),
            pltpu.VMEM((NCHUNK, G), _f32),
            pltpu.VMEM((G, 2 * HEADS), _f32),
            pltpu.VMEM((G, 2 * HEADS), _f32),
            pltpu.VMEM((G, 2 * HEADS), _f32),
            pltpu.VMEM((G, 2 * HEADS), _f32),
            pltpu.VMEM((G, HEADS), _f32),
            pltpu.VMEM((G, HEADS), _f32),
            pltpu.VMEM((ZR, HEADS), _f32),
            pltpu.VMEM_SHARED((N, HEADS), _f32),
        ] + [pltpu.SemaphoreType.DMA] * 8,
    )(src3, dst3, ew3, alpha, zrows8)


def _run_sc_aggregate(src3, dst3, p, s0, s1, nt, zrows128):
    return pl.kernel(
        _sc_aggregate,
        out_type=jax.ShapeDtypeStruct((NC, N, H), _f32),
        mesh=_MESH,
        compiler_params=pltpu.CompilerParams(needs_layout_passes=False, use_tc_tiling_on_sc=False),
        scratch_types=(
            [pltpu.VMEM((G,), _i32)] * 8
            + [pltpu.VMEM((G, HEADS), _f32)] * 6
            + [pltpu.VMEM((G * HEADS,), _f32)]
            + [pltpu.VMEM((G, H), _f32)] * 4
            + [pltpu.VMEM((ZRB, H), _f32)]
            + [pltpu.VMEM_SHARED((N, H), _f32)]
            + [pltpu.SemaphoreType.DMA] * 14
        ),
    )(src3, dst3, p, s0, s1, nt, zrows128)


def kernel(node_states, edges, edge_weights, pos_cls, W_pre, b_pre, kernels,
           attn_kernels, W_out, b_out):
    ns = node_states[0]
    src = edges[0, :, 0]
    dst = edges[0, :, 1]
    ew = edge_weights[0]

    src3 = src.reshape(NW, NCHUNK, G)
    dst3 = dst.reshape(NW, NCHUNK, G)
    ew3 = ew.reshape(NW, NCHUNK, G)

    eye = jnp.eye(HEADS, dtype=_f32)
    a12 = []
    kcat = []
    for l in range(2):
        a1 = attn_kernels[l, :, :HID, 0]
        a2 = attn_kernels[l, :, HID:, 0]
        A1 = (eye[:, None, :] * a1[:, :, None]).reshape(H, HEADS)
        A2 = (eye[:, None, :] * a2[:, :, None]).reshape(H, HEADS)
        a12.append(jnp.concatenate([A1, A2], axis=1))
        kcat.append(kernels[l].transpose(1, 0, 2).reshape(H, H))

    zrows8 = jnp.zeros((ZR, HEADS), _f32)
    zrows128 = jnp.zeros((ZRB, H), _f32)

    x0, nt0, al0 = _run_stage_a(ns, W_pre, b_pre.reshape(1, H), kcat[0], a12[0])

    p0, ss0 = _run_sc_scores(src3, dst3, ew3, al0, zrows8)
    parts0 = _run_sc_aggregate(src3, dst3, p0, ss0[0], ss0[1], nt0, zrows128)

    x1, nt1, al1 = _run_stage_b(parts0, x0, kcat[1], a12[1])

    p1, ss1 = _run_sc_scores(src3, dst3, ew3, al1, zrows8)
    parts1 = _run_sc_aggregate(src3, dst3, p1, ss1[0], ss1[1], nt1, zrows128)

    out = _run_stage_c(parts1, x1, W_out, b_out.reshape(1, OUT))
    return out[None, :, :]


# speedup vs baseline: 133.8956x; 1.0816x over previous
import functools

import jax
import jax.numpy as jnp
from jax import lax
from jax.experimental import pallas as pl
from jax.experimental.pallas import tpu as pltpu
from jax.experimental.pallas import tpu_sc as plsc

N = 10000
E = 320000
D = 128
HID = 16
HEADS = 8
H = HID * HEADS
OUT = 64

NC = 2
NS = 16
NW = NC * NS
CE = E // NW
G = 80
NCHUNK = CE // G
RPT = 624
RPT_EXTRA = N - NS * RPT
ZR = 16
ZRB = 4

_f32 = jnp.float32
_i32 = jnp.int32


TCBLK = 1000


def _tc_stage_a(ns_ref, wpre_ref, bpre_ref, k_ref, a_ref, x_ref, nt_ref, al_ref):
    x = jnp.maximum(
        jnp.dot(ns_ref[...], wpre_ref[...], preferred_element_type=_f32)
        + bpre_ref[...], 0.0)
    x_ref[...] = x
    nt = jnp.dot(x, k_ref[...], preferred_element_type=_f32)
    nt_ref[...] = nt
    al_ref[...] = jnp.dot(nt, a_ref[...], preferred_element_type=_f32)


def _tc_stage_b(parts_ref, xp_ref, k_ref, a_ref, x_ref, nt_ref, al_ref):
    xn = jnp.maximum(parts_ref[0] + parts_ref[1], 0.0) + xp_ref[...]
    x_ref[...] = xn
    nt = jnp.dot(xn, k_ref[...], preferred_element_type=_f32)
    nt_ref[...] = nt
    al_ref[...] = jnp.dot(nt, a_ref[...], preferred_element_type=_f32)


def _tc_stage_c(parts_ref, xp_ref, wout_ref, bout_ref, o_ref):
    xn = jnp.maximum(parts_ref[0] + parts_ref[1], 0.0) + xp_ref[...]
    o_ref[...] = (jnp.dot(xn, wout_ref[...], preferred_element_type=_f32)
                  + bout_ref[...])


def _run_stage_a(ns, wpre, bpre, kcat, a12):
    grid = (N // TCBLK,)
    return pl.pallas_call(
        _tc_stage_a,
        grid=grid,
        in_specs=[
            pl.BlockSpec((TCBLK, D), lambda i: (i, 0)),
            pl.BlockSpec((D, H), lambda i: (0, 0)),
            pl.BlockSpec((1, H), lambda i: (0, 0)),
            pl.BlockSpec((H, H), lambda i: (0, 0)),
            pl.BlockSpec((H, 2 * HEADS), lambda i: (0, 0)),
        ],
        out_specs=[
            pl.BlockSpec((TCBLK, H), lambda i: (i, 0)),
            pl.BlockSpec((TCBLK, H), lambda i: (i, 0)),
            pl.BlockSpec((TCBLK, 2 * HEADS), lambda i: (i, 0)),
        ],
        out_shape=[
            jax.ShapeDtypeStruct((N, H), _f32),
            jax.ShapeDtypeStruct((N, H), _f32),
            jax.ShapeDtypeStruct((N, 2 * HEADS), _f32),
        ],
    )(ns, wpre, bpre, kcat, a12)


def _run_stage_b(parts, xprev, kcat, a12):
    grid = (N // TCBLK,)
    return pl.pallas_call(
        _tc_stage_b,
        grid=grid,
        in_specs=[
            pl.BlockSpec((2, TCBLK, H), lambda i: (0, i, 0)),
            pl.BlockSpec((TCBLK, H), lambda i: (i, 0)),
            pl.BlockSpec((H, H), lambda i: (0, 0)),
            pl.BlockSpec((H, 2 * HEADS), lambda i: (0, 0)),
        ],
        out_specs=[
            pl.BlockSpec((TCBLK, H), lambda i: (i, 0)),
            pl.BlockSpec((TCBLK, H), lambda i: (i, 0)),
            pl.BlockSpec((TCBLK, 2 * HEADS), lambda i: (i, 0)),
        ],
        out_shape=[
            jax.ShapeDtypeStruct((N, H), _f32),
            jax.ShapeDtypeStruct((N, H), _f32),
            jax.ShapeDtypeStruct((N, 2 * HEADS), _f32),
        ],
    )(parts, xprev, kcat, a12)


def _run_stage_c(parts, xprev, wout, bout):
    grid = (N // TCBLK,)
    return pl.pallas_call(
        _tc_stage_c,
        grid=grid,
        in_specs=[
            pl.BlockSpec((2, TCBLK, H), lambda i: (0, i, 0)),
            pl.BlockSpec((TCBLK, H), lambda i: (i, 0)),
            pl.BlockSpec((H, OUT), lambda i: (0, 0)),
            pl.BlockSpec((1, OUT), lambda i: (0, 0)),
        ],
        out_specs=pl.BlockSpec((TCBLK, OUT), lambda i: (i, 0)),
        out_shape=jax.ShapeDtypeStruct((N, OUT), _f32),
    )(parts, xprev, wout, bout)


_MESH = plsc.VectorSubcoreMesh(core_axis_name="c", subcore_axis_name="s")
_LANE_IOTA = None


def _lanes():
    return lax.iota(_i32, 16)


def _splat(vec, j):
    dnums = lax.GatherDimensionNumbers(
        offset_dims=(), collapsed_slice_dims=(0,), start_index_map=(0,))
    idx = jnp.full((16, 1), j, dtype=_i32)
    return lax.gather(vec, idx, dnums, (1,),
                      mode=lax.GatherScatterMode.PROMISE_IN_BOUNDS)


def _sc_scores(src_hbm, dst_hbm, ew_hbm, alpha_hbm, zrows_hbm,
               p_hbm, ssum_hbm,
               idx_s, idx_d, ew_v,
               a_s0, a_s1, a_s2, a_s3, a_d0, a_d1, a_d2, a_d3,
               p_v0, p_v1, p_v2, p_v3,
               zbuf, ssum_sp,
               sga0, sga1, sga2, sga3, sgd0, sgd1, sgd2, sgd3,
               swp0, swp1, swp2, swp3, sws0, sws1, sws2, sws3):
    c = lax.axis_index("c")
    s = lax.axis_index("s")
    wid = c * NS + s
    a_s = [a_s0, a_s1, a_s2, a_s3]
    a_d = [a_d0, a_d1, a_d2, a_d3]
    p_v = [p_v0, p_v1, p_v2, p_v3]
    sga = [sga0, sga1, sga2, sga3]
    sgd = [sgd0, sgd1, sgd2, sgd3]
    swp = [swp0, swp1, swp2, swp3]
    sws = [sws0, sws1, sws2, sws3]

    pltpu.sync_copy(zrows_hbm, zbuf)

    def zero_blk(z, _):
        pltpu.sync_copy(zbuf, ssum_sp.at[pl.ds(s * RPT + z * ZR, ZR), :])
        return 0

    nzero = (RPT + jnp.where(s == NS - 1, RPT_EXTRA, 0)) // ZR
    lax.fori_loop(0, nzero, zero_blk, 0)
    plsc.subcore_barrier()

    pltpu.sync_copy(src_hbm.at[wid], idx_s)
    pltpu.sync_copy(dst_hbm.at[wid], idx_d)
    pltpu.sync_copy(ew_hbm.at[wid], ew_v)

    def issue_gathers(cg, k):
        pltpu.async_copy(alpha_hbm.at[idx_s.at[cg]], a_s[k], sga[k])
        pltpu.async_copy(alpha_hbm.at[idx_d.at[cg]], a_d[k], sgd[k])

    def wait_gathers(cg, k):
        pltpu.make_async_copy(alpha_hbm.at[idx_s.at[cg]], a_s[k], sga[k]).wait()
        pltpu.make_async_copy(alpha_hbm.at[idx_d.at[cg]], a_d[k], sgd[k]).wait()

    def wait_stores(cg, k):
        pltpu.make_async_copy(p_v[k], p_hbm.at[wid, cg], swp[k]).wait()
        pltpu.make_async_copy(p_v[k], ssum_sp.at[idx_s.at[cg]], sws[k]).wait()

    def compute(cg, k):
        def grp(i, _):
            g0 = i * 16
            rows = g0 + _lanes()
            ewv = ew_v[cg, pl.ds(g0, 16)]
            for h in range(HEADS):
                sv = plsc.load_gather(a_s[k], [rows, jnp.full((16,), h, _i32)])
                dv = plsc.load_gather(a_d[k],
                                      [rows, jnp.full((16,), h + 8, _i32)])
                sc = ewv * (sv + dv)
                sc = jnp.where(sc >= 0.0, sc, 0.01 * sc)
                sc = jnp.clip(sc, -2.0, 2.0)
                pv = jnp.exp(sc)
                plsc.store_scatter(p_v[k], [rows, jnp.full((16,), h, _i32)],
                                   pv)
            return 0

        lax.fori_loop(0, G // 16, grp, 0)
        pltpu.async_copy(p_v[k], p_hbm.at[wid, cg], swp[k])
        pltpu.async_copy(p_v[k], ssum_sp.at[idx_s.at[cg]], sws[k], add=True)

    issue_gathers(0, 0)
    issue_gathers(1, 1)

    def quad(i, _):
        for kk in range(4):
            cg = i * 4 + kk
            if kk >= 2:
                wait_stores(cg - 2, (kk - 2) % 4)
            else:
                @pl.when(i > 0)
                def _():
                    wait_stores(cg, (kk + 2) % 4)

            @pl.when(cg + 2 < NCHUNK)
            def _():
                issue_gathers(cg + 2, (kk + 2) % 4)

            wait_gathers(cg, kk)
            compute(cg, kk)
        return 0

    lax.fori_loop(0, NCHUNK // 4, quad, 0)
    wait_stores(NCHUNK - 3, 2)
    wait_gathers(NCHUNK - 1, 0)
    compute(NCHUNK - 1, 0)
    wait_stores(NCHUNK - 2, 3)
    wait_stores(NCHUNK - 1, 0)
    plsc.subcore_barrier()
    pltpu.sync_copy(ssum_sp.at[pl.ds(s * RPT, RPT), :],
                    ssum_hbm.at[c, pl.ds(s * RPT, RPT), :])

    @pl.when(s == NS - 1)
    def _():
        pltpu.sync_copy(ssum_sp.at[pl.ds(NS * RPT, RPT_EXTRA), :],
                        ssum_hbm.at[c, pl.ds(NS * RPT, RPT_EXTRA), :])


def _sc_aggregate(src_hbm, dst_hbm, p_hbm, s01_hbm, nt_hbm, zrows_hbm,
                  out_hbm,
                  ixg_s0, ixg_s1, ixg_s2, ixg_s3,
                  ixg_d0, ixg_d1, ixg_d2, ixg_d3,
                  ixw0, ixw1, ixw2, ixw3,
                  p_v0, p_v1, p_v2, p_v3,
                  sv0, sv1, sv2, sv3,
                  nrm_f, rin0, rin1, rin2, rin3, zbuf, out_sp,
                  sig0, sig1, sig2, sig3, siw0, siw1, siw2, siw3,
                  sr0, sr1, sr2, sr3, ssv0, ssv1, ssv2, ssv3,
                  sp0, sp1, sp2, sp3, sw0, sw1, sw2, sw3):
    c = lax.axis_index("c")
    s = lax.axis_index("s")
    wid = c * NS + s
    ixg_s = [ixg_s0, ixg_s1, ixg_s2, ixg_s3]
    ixg_d = [ixg_d0, ixg_d1, ixg_d2, ixg_d3]
    ixw = [ixw0, ixw1, ixw2, ixw3]
    p_v = [p_v0, p_v1, p_v2, p_v3]
    sv = [sv0, sv1, sv2, sv3]
    rin = [rin0, rin1, rin2, rin3]
    sig = [sig0, sig1, sig2, sig3]
    siw = [siw0, siw1, siw2, siw3]
    sr = [sr0, sr1, sr2, sr3]
    ssv = [ssv0, ssv1, ssv2, ssv3]
    sp = [sp0, sp1, sp2, sp3]
    sw = [sw0, sw1, sw2, sw3]

    pltpu.sync_copy(zrows_hbm, zbuf)

    def zero_blk(z, _):
        pltpu.sync_copy(zbuf, out_sp.at[pl.ds(s * RPT + z * ZRB, ZRB), :])
        return 0

    nzero = (RPT + jnp.where(s == NS - 1, RPT_EXTRA, 0)) // ZRB
    lax.fori_loop(0, nzero, zero_blk, 0)
    plsc.subcore_barrier()

    def issue_idx_g(cg, k):
        pltpu.async_copy(src_hbm.at[wid, cg], ixg_s[k], sig[k])
        pltpu.async_copy(dst_hbm.at[wid, cg], ixg_d[k], sig[k])

    def wait_idx_g(cg, k):
        pltpu.make_async_copy(src_hbm.at[wid, cg], ixg_s[k], sig[k]).wait()
        pltpu.make_async_copy(dst_hbm.at[wid, cg], ixg_d[k], sig[k]).wait()

    def issue_idx_w(cg, k):
        pltpu.async_copy(src_hbm.at[wid, cg], ixw[k], siw[k])

    def wait_idx_w(cg, k):
        pltpu.make_async_copy(src_hbm.at[wid, cg], ixw[k], siw[k]).wait()

    def issue_gathers(cg, k):
        pltpu.async_copy(nt_hbm.at[ixg_d[k]], rin[k], sr[k])
        pltpu.async_copy(s01_hbm.at[ixg_s[k]], sv[k], ssv[k])
        pltpu.async_copy(p_hbm.at[wid, cg], p_v[k], sp[k])

    def wait_gathers(cg, k):
        pltpu.make_async_copy(nt_hbm.at[ixg_d[k]], rin[k], sr[k]).wait()
        pltpu.make_async_copy(s01_hbm.at[ixg_s[k]], sv[k], ssv[k]).wait()
        pltpu.make_async_copy(p_hbm.at[wid, cg], p_v[k], sp[k]).wait()

    def wait_store(k):
        pltpu.make_async_copy(rin[k], out_sp.at[ixw[k]], sw[k]).wait()

    def compute(cg, k):
        def nrm_grp(i, _):
            flat = i * 16 + _lanes()
            r = flat // 8
            col = flat % 8
            pv = plsc.load_gather(p_v[k], [r, col])
            t0 = plsc.load_gather(sv[k], [r, col])
            t1 = plsc.load_gather(sv[k], [r, col + 8])
            plsc.store_scatter(nrm_f, [flat], pv / (t0 + t1))
            return 0

        lax.fori_loop(0, (G * HEADS) // 16, nrm_grp, 0)

        def pair(i, _):
            nv = nrm_f[pl.ds(i * 16, 16)]
            e0 = i * 2
            for j in range(HEADS):
                sl0 = _splat(nv, j)
                rin[k][e0, pl.ds(j * HID, HID)] = (
                    rin[k][e0, pl.ds(j * HID, HID)] * sl0)
                sl1 = _splat(nv, 8 + j)
                rin[k][e0 + 1, pl.ds(j * HID, HID)] = (
                    rin[k][e0 + 1, pl.ds(j * HID, HID)] * sl1)
            return 0

        lax.fori_loop(0, G // 2, pair, 0)
        wait_idx_w(cg, k)
        pltpu.async_copy(rin[k], out_sp.at[ixw[k]], sw[k], add=True)

    issue_idx_g(0, 0)
    issue_idx_g(1, 1)
    issue_idx_g(2, 2)
    issue_idx_w(0, 0)
    issue_idx_w(1, 1)
    wait_idx_g(0, 0)
    issue_gathers(0, 0)
    wait_idx_g(1, 1)
    issue_gathers(1, 1)

    def quad(i, _):
        for kk in range(4):
            cg = i * 4 + kk
            if kk >= 2:
                wait_store((kk - 2) % 4)
            else:
                @pl.when(i > 0)
                def _():
                    wait_store((kk + 2) % 4)

            @pl.when(cg + 2 < NCHUNK)
            def _():
                issue_idx_w(cg + 2, (kk + 2) % 4)

            @pl.when(cg + 3 < NCHUNK)
            def _():
                issue_idx_g(cg + 3, (kk + 3) % 4)

            @pl.when(cg + 2 < NCHUNK)
            def _():
                wait_idx_g(cg + 2, (kk + 2) % 4)
                issue_gathers(cg + 2, (kk + 2) % 4)

            wait_gathers(cg, kk)
            compute(cg, kk)
        return 0

    lax.fori_loop(0, NCHUNK // 4, quad, 0)
    wait_store(2)
    wait_gathers(NCHUNK - 1, 0)
    compute(NCHUNK - 1, 0)
    wait_store(3)
    wait_store(0)
    plsc.subcore_barrier()
    pltpu.sync_copy(out_sp.at[pl.ds(s * RPT, RPT), :],
                    out_hbm.at[c, pl.ds(s * RPT, RPT), :])

    @pl.when(s == NS - 1)
    def _():
        pltpu.sync_copy(out_sp.at[pl.ds(NS * RPT, RPT_EXTRA), :],
                        out_hbm.at[c, pl.ds(NS * RPT, RPT_EXTRA), :])


def _run_sc_scores(src3, dst3, ew3, alpha, zrows8):
    return pl.kernel(
        _sc_scores,
        out_type=[
            jax.ShapeDtypeStruct((NW, NCHUNK, G, HEADS), _f32),
            jax.ShapeDtypeStruct((NC, N, HEADS), _f32),
        ],
        mesh=_MESH,
        compiler_params=pltpu.CompilerParams(
            needs_layout_passes=False, use_tc_tiling_on_sc=False),
        scratch_types=(
            [pltpu.VMEM((NCHUNK, G), _i32)] * 2
            + [pltpu.VMEM((NCHUNK, G), _f32)]
            + [pltpu.VMEM((G, 2 * HEADS), _f32)] * 8
            + [pltpu.VMEM((G, HEADS), _f32)] * 4
            + [pltpu.VMEM((ZR, HEADS), _f32)]
            + [pltpu.VMEM_SHARED((N, HEADS), _f32)]
            + [pltpu.SemaphoreType.DMA] * 16
        ),
    )(src3, dst3, ew3, alpha, zrows8)


def _run_sc_aggregate(src3, dst3, p, s01, nt, zrows128):
    return pl.kernel(
        _sc_aggregate,
        out_type=jax.ShapeDtypeStruct((NC, N, H), _f32),
        mesh=_MESH,
        compiler_params=pltpu.CompilerParams(
            needs_layout_passes=False, use_tc_tiling_on_sc=False),
        scratch_types=(
            [pltpu.VMEM((G,), _i32)] * 12
            + [pltpu.VMEM((G, HEADS), _f32)] * 4
            + [pltpu.VMEM((G, 2 * HEADS), _f32)] * 4
            + [pltpu.VMEM((G * HEADS,), _f32)]
            + [pltpu.VMEM((G, H), _f32)] * 4
            + [pltpu.VMEM((ZRB, H), _f32)]
            + [pltpu.VMEM_SHARED((N, H), _f32)]
            + [pltpu.SemaphoreType.DMA] * 24
        ),
    )(src3, dst3, p, s01, nt, zrows128)


def kernel(node_states, edges, edge_weights, pos_cls, W_pre, b_pre, kernels,
           attn_kernels, W_out, b_out):
    ns = node_states[0]
    src = edges[0, :, 0]
    dst = edges[0, :, 1]
    ew = edge_weights[0]

    src3 = src.reshape(NW, NCHUNK, G)
    dst3 = dst.reshape(NW, NCHUNK, G)
    ew3 = ew.reshape(NW, NCHUNK, G)

    eye = jnp.eye(HEADS, dtype=_f32)
    a12 = []
    kcat = []
    for l in range(2):
        a1 = attn_kernels[l, :, :HID, 0]
        a2 = attn_kernels[l, :, HID:, 0]
        A1 = (eye[:, None, :] * a1[:, :, None]).reshape(H, HEADS)
        A2 = (eye[:, None, :] * a2[:, :, None]).reshape(H, HEADS)
        a12.append(jnp.concatenate([A1, A2], axis=1))
        kcat.append(kernels[l].transpose(1, 0, 2).reshape(H, H))

    zrows8 = jnp.zeros((ZR, HEADS), _f32)
    zrows128 = jnp.zeros((ZRB, H), _f32)

    x0, nt0, al0 = _run_stage_a(ns, W_pre, b_pre.reshape(1, H), kcat[0], a12[0])

    p0, ss0 = _run_sc_scores(src3, dst3, ew3, al0, zrows8)
    s01_0 = ss0.transpose(1, 0, 2).reshape(N, 2 * HEADS)
    parts0 = _run_sc_aggregate(src3, dst3, p0, s01_0, nt0, zrows128)

    x1, nt1, al1 = _run_stage_b(parts0, x0, kcat[1], a12[1])

    p1, ss1 = _run_sc_scores(src3, dst3, ew3, al1, zrows8)
    s01_1 = ss1.transpose(1, 0, 2).reshape(N, 2 * HEADS)
    parts1 = _run_sc_aggregate(src3, dst3, p1, s01_1, nt1, zrows128)

    out = _run_stage_c(parts1, x1, W_out, b_out.reshape(1, OUT))
    return out[None, :, :]
